# baseline (speedup 1.0000x reference)
import functools

import jax
import jax.numpy as jnp
from jax import lax
from jax.experimental import pallas as pl
from jax.experimental.pallas import tpu as pltpu
from jax.experimental.pallas import tpu_sc as plsc

NC = 2
NS = 16
LN = 16
CK = 128
BN = 256


def _mesh():
  return plsc.VectorSubcoreMesh(
      core_axis_name="c", subcore_axis_name="s", num_cores=NC,
      num_subcores=NS)


def _deg_call(dst2d, npad):
  er = dst2d.shape[0]
  rpt = er // (NC * NS)
  npt = npad // NS

  @functools.partial(
      pl.kernel,
      out_type=jax.ShapeDtypeStruct((NC, npad), jnp.float32),
      mesh=_mesh(),
      scratch_types=[
          pltpu.VMEM((rpt, CK), jnp.int32),
          pltpu.VMEM((CK,), jnp.float32),
          pltpu.VMEM((npt,), jnp.float32),
          pltpu.VMEM_SHARED((npad,), jnp.float32),
      ])
  def k(dst_hbm, out_hbm, idx_v, ones_v, zero_v, acc):
    c = lax.axis_index("c")
    s = lax.axis_index("s")
    tid = c * NS + s

    def fill_zero(r, carry):
      zero_v[pl.ds(r * LN, LN)] = jnp.zeros((LN,), jnp.float32)
      return carry

    lax.fori_loop(0, npt // LN, fill_zero, 0)
    for j in range(CK // LN):
      ones_v[pl.ds(j * LN, LN)] = jnp.ones((LN,), jnp.float32)

    pltpu.sync_copy(zero_v, acc.at[pl.ds(s * npt, npt)])
    pltpu.sync_copy(dst_hbm.at[pl.ds(tid * rpt, rpt)], idx_v)
    plsc.subcore_barrier()

    def deg_body(j, carry):
      pltpu.sync_copy(ones_v, acc.at[idx_v.at[j]], add=True)
      return carry

    lax.fori_loop(0, rpt, deg_body, 0)
    plsc.subcore_barrier()
    pltpu.sync_copy(acc.at[pl.ds(s * npt, npt)],
                    out_hbm.at[c, pl.ds(s * npt, npt)])

  return k(dst2d)


WIN = 40


def _edge_pipe(h_hbm, src_hbm, dst_hbm, base, nwin,
               src_v, dst_v, rows_a, rows_b, acc, sem_g, sem_s):
  def win_body(w, carry):
    pltpu.sync_copy(src_hbm.at[pl.ds(base + w * WIN, WIN)], src_v)
    pltpu.sync_copy(dst_hbm.at[pl.ds(base + w * WIN, WIN)], dst_v)
    pltpu.async_copy(h_hbm.at[src_v.at[0]], rows_a, sem_g)

    def body(j2, c2):
      j = j2 * 2
      for b, cur, nxt in ((0, rows_a, rows_b), (1, rows_b, rows_a)):
        jj = j + b
        pltpu.make_async_copy(h_hbm.at[src_v.at[jj]], cur, sem_g).wait()

        @pl.when(jj > 0)
        def _():
          pltpu.make_async_copy(nxt, acc.at[dst_v.at[jj]], sem_s).wait()

        @pl.when(jj + 1 < WIN)
        def _():
          pltpu.async_copy(h_hbm.at[src_v.at[jj + 1]], nxt, sem_g)

        pltpu.async_copy(cur, acc.at[dst_v.at[jj]], sem_s, add=True)
      return c2

    lax.fori_loop(0, WIN // 2, body, 0)
    pltpu.make_async_copy(rows_b, acc.at[dst_v.at[WIN - 1]], sem_s).wait()
    return carry

  lax.fori_loop(0, nwin, win_body, 0)


def _agg1_call(h1p, src2d, dst2d, npad):
  er = src2d.shape[0]
  rpt = er // (NC * NS)
  npt = npad // NS
  h = h1p.shape[1]

  @functools.partial(
      pl.kernel,
      out_type=jax.ShapeDtypeStruct((NC, npad, h), jnp.float32),
      mesh=_mesh(),
      scratch_types=[
          pltpu.VMEM((WIN, CK), jnp.int32),
          pltpu.VMEM((WIN, CK), jnp.int32),
          pltpu.VMEM((CK, h), jnp.float32),
          pltpu.VMEM((CK, h), jnp.float32),
          pltpu.VMEM_SHARED((npad, h), jnp.float32),
          pltpu.SemaphoreType.DMA,
          pltpu.SemaphoreType.DMA,
      ])
  def k(h_hbm, src_hbm, dst_hbm, out_hbm, src_v, dst_v, rows_a, rows_b,
        acc, sem_g, sem_s):
    c = lax.axis_index("c")
    s = lax.axis_index("s")
    tid = c * NS + s
    pltpu.sync_copy(h_hbm.at[pl.ds(s * npt, npt)],
                    acc.at[pl.ds(s * npt, npt)])
    plsc.subcore_barrier()
    _edge_pipe(h_hbm, src_hbm, dst_hbm, tid * rpt, rpt // WIN,
               src_v, dst_v, rows_a, rows_b, acc, sem_g, sem_s)
    plsc.subcore_barrier()
    pltpu.sync_copy(acc.at[pl.ds(s * npt, npt)],
                    out_hbm.at[c, pl.ds(s * npt, npt)])

  return k(h1p, src2d, dst2d)


def _agg2_call(h2a, h2b, src2d, dst2d, npad):
  er = src2d.shape[0]
  rpt = er // NS
  npt = npad // NS
  h = h2a.shape[1]

  @functools.partial(
      pl.kernel,
      out_type=(jax.ShapeDtypeStruct((npad, h), jnp.float32),
                jax.ShapeDtypeStruct((npad, h), jnp.float32)),
      mesh=_mesh(),
      scratch_types=[
          pltpu.VMEM((WIN, CK), jnp.int32),
          pltpu.VMEM((WIN, CK), jnp.int32),
          pltpu.VMEM((CK, h), jnp.float32),
          pltpu.VMEM((CK, h), jnp.float32),
          pltpu.VMEM_SHARED((npad, h), jnp.float32),
          pltpu.SemaphoreType.DMA,
          pltpu.SemaphoreType.DMA,
      ])
  def k(ha_hbm, hb_hbm, src_hbm, dst_hbm, outa_hbm, outb_hbm,
        src_v, dst_v, rows_a, rows_b, acc, sem_g, sem_s):
    c = lax.axis_index("c")
    s = lax.axis_index("s")
    for ci, h_hbm, out_hbm in ((0, ha_hbm, outa_hbm), (1, hb_hbm, outb_hbm)):

      @pl.when(c == ci)
      def _():
        pltpu.sync_copy(h_hbm.at[pl.ds(s * npt, npt)],
                        acc.at[pl.ds(s * npt, npt)])
        plsc.subcore_barrier()
        _edge_pipe(h_hbm, src_hbm, dst_hbm, s * rpt, rpt // WIN,
                   src_v, dst_v, rows_a, rows_b, acc, sem_g, sem_s)
        plsc.subcore_barrier()
        pltpu.sync_copy(acc.at[pl.ds(s * npt, npt)],
                        out_hbm.at[pl.ds(s * npt, npt)])

  return k(h2a, h2b, src2d, dst2d)


def _tc1_call(xp, w1, d0, d1):
  npad, din = xp.shape
  dh = w1.shape[1]
  grid = (npad // BN,)

  def body(x_ref, w_ref, d0_ref, d1_ref, o_ref):
    dinv = lax.rsqrt(1.0 + d0_ref[...] + d1_ref[...])
    o_ref[...] = jnp.dot(x_ref[...], w_ref[...],
                         preferred_element_type=jnp.float32) * dinv[:, None]

  return pl.pallas_call(
      body,
      grid=grid,
      in_specs=[
          pl.BlockSpec((BN, din), lambda i: (i, 0)),
          pl.BlockSpec((din, dh), lambda i: (0, 0)),
          pl.BlockSpec((BN,), lambda i: (i,)),
          pl.BlockSpec((BN,), lambda i: (i,)),
      ],
      out_specs=pl.BlockSpec((BN, dh), lambda i: (i, 0)),
      out_shape=jax.ShapeDtypeStruct((npad, dh), jnp.float32),
  )(xp, w1, d0, d1)


def _tc2_call(a1a, a1b, h1p, d0, d1, b1, w2):
  npad, dh = h1p.shape
  dout = w2.shape[1]
  hh = dout // 2
  grid = (npad // BN,)

  def body(aa_ref, ab_ref, hp_ref, d0_ref, d1_ref, b1_ref, w_ref,
           oa_ref, ob_ref):
    dinv = lax.rsqrt(1.0 + d0_ref[...] + d1_ref[...])
    agg = aa_ref[...] + ab_ref[...] - hp_ref[...]
    t = jnp.maximum(agg * dinv[:, None] + b1_ref[...][None, :], 0.0)
    h2p = jnp.dot(t, w_ref[...],
                  preferred_element_type=jnp.float32) * dinv[:, None]
    oa_ref[...] = h2p[:, :hh]
    ob_ref[...] = h2p[:, hh:]

  return pl.pallas_call(
      body,
      grid=grid,
      in_specs=[
          pl.BlockSpec((BN, dh), lambda i: (i, 0)),
          pl.BlockSpec((BN, dh), lambda i: (i, 0)),
          pl.BlockSpec((BN, dh), lambda i: (i, 0)),
          pl.BlockSpec((BN,), lambda i: (i,)),
          pl.BlockSpec((BN,), lambda i: (i,)),
          pl.BlockSpec((dh,), lambda i: (0,)),
          pl.BlockSpec((dh, dout), lambda i: (0, 0)),
      ],
      out_specs=[
          pl.BlockSpec((BN, hh), lambda i: (i, 0)),
          pl.BlockSpec((BN, hh), lambda i: (i, 0)),
      ],
      out_shape=(jax.ShapeDtypeStruct((npad, hh), jnp.float32),
                 jax.ShapeDtypeStruct((npad, hh), jnp.float32)),
  )(a1a, a1b, h1p, d0, d1, b1, w2)


def _tc3_call(a2a, a2b, d0, d1, b2, batchp, ngraphs):
  npad, hh = a2a.shape
  dout = 2 * hh
  grid = (npad // BN,)
  nsteps = npad // BN

  def body(aa_ref, ab_ref, d0_ref, d1_ref, b2_ref, bt_ref, o_ref,
           accp, cnt):
    i = pl.program_id(0)

    @pl.when(i == 0)
    def _():
      accp[...] = jnp.zeros_like(accp)
      cnt[...] = jnp.zeros_like(cnt)

    dinv = lax.rsqrt(1.0 + d0_ref[...] + d1_ref[...])
    h2 = (jnp.concatenate([aa_ref[...], ab_ref[...]], axis=1)
          * dinv[:, None] + b2_ref[...][None, :])
    gids = lax.broadcasted_iota(jnp.int32, (1, ngraphs), 1)
    oh = (bt_ref[...][:, None] == gids).astype(jnp.float32)
    accp[...] += lax.dot_general(oh, h2, (((0,), (0,)), ((), ())),
                                 preferred_element_type=jnp.float32)
    cnt[...] += jnp.sum(oh, axis=0)

    @pl.when(i == nsteps - 1)
    def _():
      pooled = accp[...] / jnp.maximum(cnt[...], 1.0)[:, None]
      m = jnp.max(pooled, axis=1, keepdims=True)
      lse = jnp.log(jnp.sum(jnp.exp(pooled - m), axis=1, keepdims=True))
      o_ref[...] = pooled - m - lse

  return pl.pallas_call(
      body,
      grid=grid,
      in_specs=[
          pl.BlockSpec((BN, hh), lambda i: (i, 0)),
          pl.BlockSpec((BN, hh), lambda i: (i, 0)),
          pl.BlockSpec((BN,), lambda i: (i,)),
          pl.BlockSpec((BN,), lambda i: (i,)),
          pl.BlockSpec((dout,), lambda i: (0,)),
          pl.BlockSpec((BN,), lambda i: (i,)),
      ],
      out_specs=pl.BlockSpec((ngraphs, dout), lambda i: (0, 0)),
      out_shape=jax.ShapeDtypeStruct((ngraphs, dout), jnp.float32),
      scratch_shapes=[
          pltpu.VMEM((ngraphs, dout), jnp.float32),
          pltpu.VMEM((ngraphs,), jnp.float32),
      ],
  )(a2a, a2b, d0, d1, b2, batchp)


def kernel(x, edge_index, batch, W1, b1, W2, b2):
  n = x.shape[0]
  e = edge_index.shape[1]
  ngraphs = 64

  npad = ((n + 64 + 255) // 256) * 256
  ecell = NC * NS * CK * WIN
  epad = ((e + ecell - 1) // ecell) * ecell
  spread = 64
  pad_idx = n + (jnp.arange(epad - e, dtype=jnp.int32) % spread)
  src2d = jnp.concatenate([edge_index[0], pad_idx]).reshape(epad // CK, CK)
  dst2d = jnp.concatenate([edge_index[1], pad_idx]).reshape(epad // CK, CK)
  xp = jnp.pad(x, ((0, npad - n), (0, 0)))
  batchp = jnp.pad(batch, (0, npad - n), constant_values=ngraphs)

  degp = _deg_call(dst2d, npad)
  d0, d1 = degp[0], degp[1]
  h1p = _tc1_call(xp, W1, d0, d1)
  a1 = _agg1_call(h1p, src2d, dst2d, npad)
  h2a, h2b = _tc2_call(a1[0], a1[1], h1p, d0, d1, b1, W2)
  a2a, a2b = _agg2_call(h2a, h2b, src2d, dst2d, npad)
  return _tc3_call(a2a, a2b, d0, d1, b2, batchp, ngraphs)


# speedup vs baseline: 1.0113x; 1.0113x over previous
import functools

import jax
import jax.numpy as jnp
from jax import lax
from jax.experimental import pallas as pl
from jax.experimental.pallas import tpu as pltpu
from jax.experimental.pallas import tpu_sc as plsc

NC = 2
NS = 16
LN = 16
CK = 64
BN = 256


def _mesh():
  return plsc.VectorSubcoreMesh(
      core_axis_name="c", subcore_axis_name="s", num_cores=NC,
      num_subcores=NS)


def _deg_call(dst2d, npad):
  er = dst2d.shape[0]
  rpt = er // (NC * NS)
  npt = npad // NS

  @functools.partial(
      pl.kernel,
      out_type=jax.ShapeDtypeStruct((NC, npad), jnp.float32),
      mesh=_mesh(),
      scratch_types=[
          pltpu.VMEM((rpt, CK), jnp.int32),
          pltpu.VMEM((CK,), jnp.float32),
          pltpu.VMEM((npt,), jnp.float32),
          pltpu.VMEM_SHARED((npad,), jnp.float32),
      ])
  def k(dst_hbm, out_hbm, idx_v, ones_v, zero_v, acc):
    c = lax.axis_index("c")
    s = lax.axis_index("s")
    tid = c * NS + s

    def fill_zero(r, carry):
      zero_v[pl.ds(r * LN, LN)] = jnp.zeros((LN,), jnp.float32)
      return carry

    lax.fori_loop(0, npt // LN, fill_zero, 0)
    for j in range(CK // LN):
      ones_v[pl.ds(j * LN, LN)] = jnp.ones((LN,), jnp.float32)

    pltpu.sync_copy(zero_v, acc.at[pl.ds(s * npt, npt)])
    pltpu.sync_copy(dst_hbm.at[pl.ds(tid * rpt, rpt)], idx_v)
    plsc.subcore_barrier()

    def deg_body(j, carry):
      pltpu.sync_copy(ones_v, acc.at[idx_v.at[j]], add=True)
      return carry

    lax.fori_loop(0, rpt, deg_body, 0)
    plsc.subcore_barrier()
    pltpu.sync_copy(acc.at[pl.ds(s * npt, npt)],
                    out_hbm.at[c, pl.ds(s * npt, npt)])

  return k(dst2d)


WIN = 40


NBUF = 4


def _edge_pipe(h_hbm, src_hbm, dst_hbm, base, nwin,
               src_v, dst_v, rows, gsems, ssems, acc):
  def win_body(w, carry):
    pltpu.sync_copy(src_hbm.at[pl.ds(base + w * WIN, WIN)], src_v)
    pltpu.sync_copy(dst_hbm.at[pl.ds(base + w * WIN, WIN)], dst_v)
    pltpu.async_copy(h_hbm.at[src_v.at[0]], rows[0], gsems[0])
    pltpu.async_copy(h_hbm.at[src_v.at[1]], rows[1], gsems[1])

    def body(j4, c4):
      j = j4 * NBUF
      for b in range(NBUF):
        jj = j + b
        bn = (b + 2) % NBUF
        pltpu.make_async_copy(h_hbm.at[src_v.at[jj]], rows[b],
                              gsems[b]).wait()

        @pl.when(jj >= 2)
        def _():
          pltpu.make_async_copy(rows[bn], acc.at[dst_v.at[jj]],
                                ssems[bn]).wait()

        @pl.when(jj + 2 < WIN)
        def _():
          pltpu.async_copy(h_hbm.at[src_v.at[jj + 2]], rows[bn], gsems[bn])

        pltpu.async_copy(rows[b], acc.at[dst_v.at[jj]], ssems[b], add=True)
      return c4

    lax.fori_loop(0, WIN // NBUF, body, 0)
    for jj in (WIN - 2, WIN - 1):
      b = jj % NBUF
      pltpu.make_async_copy(rows[b], acc.at[dst_v.at[jj]], ssems[b]).wait()
    return carry

  lax.fori_loop(0, nwin, win_body, 0)


def _agg1_call(h1p, src2d, dst2d, npad):
  er = src2d.shape[0]
  rpt = er // (NC * NS)
  npt = npad // NS
  h = h1p.shape[1]

  @functools.partial(
      pl.kernel,
      out_type=jax.ShapeDtypeStruct((NC, npad, h), jnp.float32),
      mesh=_mesh(),
      scratch_types=[
          pltpu.VMEM((WIN, CK), jnp.int32),
          pltpu.VMEM((WIN, CK), jnp.int32),
      ] + [pltpu.VMEM((CK, h), jnp.float32)] * NBUF
        + [pltpu.VMEM_SHARED((npad, h), jnp.float32)]
        + [pltpu.SemaphoreType.DMA] * (2 * NBUF))
  def k(h_hbm, src_hbm, dst_hbm, out_hbm, src_v, dst_v, r0, r1, r2, r3,
        acc, g0, g1, g2, g3, s0, s1, s2, s3):
    rows = (r0, r1, r2, r3)
    gsems = (g0, g1, g2, g3)
    ssems = (s0, s1, s2, s3)
    c = lax.axis_index("c")
    s = lax.axis_index("s")
    tid = c * NS + s
    pltpu.sync_copy(h_hbm.at[pl.ds(s * npt, npt)],
                    acc.at[pl.ds(s * npt, npt)])
    plsc.subcore_barrier()
    _edge_pipe(h_hbm, src_hbm, dst_hbm, tid * rpt, rpt // WIN,
               src_v, dst_v, rows, gsems, ssems, acc)
    plsc.subcore_barrier()
    pltpu.sync_copy(acc.at[pl.ds(s * npt, npt)],
                    out_hbm.at[c, pl.ds(s * npt, npt)])

  return k(h1p, src2d, dst2d)


def _agg2_call(h2a, h2b, src2d, dst2d, npad):
  er = src2d.shape[0]
  rpt = er // NS
  npt = npad // NS
  h = h2a.shape[1]

  @functools.partial(
      pl.kernel,
      out_type=(jax.ShapeDtypeStruct((npad, h), jnp.float32),
                jax.ShapeDtypeStruct((npad, h), jnp.float32)),
      mesh=_mesh(),
      scratch_types=[
          pltpu.VMEM((WIN, CK), jnp.int32),
          pltpu.VMEM((WIN, CK), jnp.int32),
      ] + [pltpu.VMEM((CK, h), jnp.float32)] * NBUF
        + [pltpu.VMEM_SHARED((npad, h), jnp.float32)]
        + [pltpu.SemaphoreType.DMA] * (2 * NBUF))
  def k(ha_hbm, hb_hbm, src_hbm, dst_hbm, outa_hbm, outb_hbm,
        src_v, dst_v, r0, r1, r2, r3,
        acc, g0, g1, g2, g3, s0, s1, s2, s3):
    rows = (r0, r1, r2, r3)
    gsems = (g0, g1, g2, g3)
    ssems = (s0, s1, s2, s3)
    c = lax.axis_index("c")
    s = lax.axis_index("s")
    for ci, h_hbm, out_hbm in ((0, ha_hbm, outa_hbm), (1, hb_hbm, outb_hbm)):

      @pl.when(c == ci)
      def _():
        pltpu.sync_copy(h_hbm.at[pl.ds(s * npt, npt)],
                        acc.at[pl.ds(s * npt, npt)])
        plsc.subcore_barrier()
        _edge_pipe(h_hbm, src_hbm, dst_hbm, s * rpt, rpt // WIN,
                   src_v, dst_v, rows, gsems, ssems, acc)
        plsc.subcore_barrier()
        pltpu.sync_copy(acc.at[pl.ds(s * npt, npt)],
                        out_hbm.at[pl.ds(s * npt, npt)])

  return k(h2a, h2b, src2d, dst2d)


def _tc1_call(xp, w1, d0, d1):
  npad, din = xp.shape
  dh = w1.shape[1]
  grid = (npad // BN,)

  def body(x_ref, w_ref, d0_ref, d1_ref, o_ref):
    dinv = lax.rsqrt(1.0 + d0_ref[...] + d1_ref[...])
    o_ref[...] = jnp.dot(x_ref[...], w_ref[...],
                         preferred_element_type=jnp.float32) * dinv[:, None]

  return pl.pallas_call(
      body,
      grid=grid,
      in_specs=[
          pl.BlockSpec((BN, din), lambda i: (i, 0)),
          pl.BlockSpec((din, dh), lambda i: (0, 0)),
          pl.BlockSpec((BN,), lambda i: (i,)),
          pl.BlockSpec((BN,), lambda i: (i,)),
      ],
      out_specs=pl.BlockSpec((BN, dh), lambda i: (i, 0)),
      out_shape=jax.ShapeDtypeStruct((npad, dh), jnp.float32),
  )(xp, w1, d0, d1)


def _tc2_call(a1a, a1b, h1p, d0, d1, b1, w2):
  npad, dh = h1p.shape
  dout = w2.shape[1]
  hh = dout // 2
  grid = (npad // BN,)

  def body(aa_ref, ab_ref, hp_ref, d0_ref, d1_ref, b1_ref, w_ref,
           oa_ref, ob_ref):
    dinv = lax.rsqrt(1.0 + d0_ref[...] + d1_ref[...])
    agg = aa_ref[...] + ab_ref[...] - hp_ref[...]
    t = jnp.maximum(agg * dinv[:, None] + b1_ref[...][None, :], 0.0)
    h2p = jnp.dot(t, w_ref[...],
                  preferred_element_type=jnp.float32) * dinv[:, None]
    oa_ref[...] = h2p[:, :hh]
    ob_ref[...] = h2p[:, hh:]

  return pl.pallas_call(
      body,
      grid=grid,
      in_specs=[
          pl.BlockSpec((BN, dh), lambda i: (i, 0)),
          pl.BlockSpec((BN, dh), lambda i: (i, 0)),
          pl.BlockSpec((BN, dh), lambda i: (i, 0)),
          pl.BlockSpec((BN,), lambda i: (i,)),
          pl.BlockSpec((BN,), lambda i: (i,)),
          pl.BlockSpec((dh,), lambda i: (0,)),
          pl.BlockSpec((dh, dout), lambda i: (0, 0)),
      ],
      out_specs=[
          pl.BlockSpec((BN, hh), lambda i: (i, 0)),
          pl.BlockSpec((BN, hh), lambda i: (i, 0)),
      ],
      out_shape=(jax.ShapeDtypeStruct((npad, hh), jnp.float32),
                 jax.ShapeDtypeStruct((npad, hh), jnp.float32)),
  )(a1a, a1b, h1p, d0, d1, b1, w2)


def _tc3_call(a2a, a2b, d0, d1, b2, batchp, ngraphs):
  npad, hh = a2a.shape
  dout = 2 * hh
  grid = (npad // BN,)
  nsteps = npad // BN

  def body(aa_ref, ab_ref, d0_ref, d1_ref, b2_ref, bt_ref, o_ref,
           accp, cnt):
    i = pl.program_id(0)

    @pl.when(i == 0)
    def _():
      accp[...] = jnp.zeros_like(accp)
      cnt[...] = jnp.zeros_like(cnt)

    dinv = lax.rsqrt(1.0 + d0_ref[...] + d1_ref[...])
    h2 = (jnp.concatenate([aa_ref[...], ab_ref[...]], axis=1)
          * dinv[:, None] + b2_ref[...][None, :])
    gids = lax.broadcasted_iota(jnp.int32, (1, ngraphs), 1)
    oh = (bt_ref[...][:, None] == gids).astype(jnp.float32)
    accp[...] += lax.dot_general(oh, h2, (((0,), (0,)), ((), ())),
                                 preferred_element_type=jnp.float32)
    cnt[...] += jnp.sum(oh, axis=0)

    @pl.when(i == nsteps - 1)
    def _():
      pooled = accp[...] / jnp.maximum(cnt[...], 1.0)[:, None]
      m = jnp.max(pooled, axis=1, keepdims=True)
      lse = jnp.log(jnp.sum(jnp.exp(pooled - m), axis=1, keepdims=True))
      o_ref[...] = pooled - m - lse

  return pl.pallas_call(
      body,
      grid=grid,
      in_specs=[
          pl.BlockSpec((BN, hh), lambda i: (i, 0)),
          pl.BlockSpec((BN, hh), lambda i: (i, 0)),
          pl.BlockSpec((BN,), lambda i: (i,)),
          pl.BlockSpec((BN,), lambda i: (i,)),
          pl.BlockSpec((dout,), lambda i: (0,)),
          pl.BlockSpec((BN,), lambda i: (i,)),
      ],
      out_specs=pl.BlockSpec((ngraphs, dout), lambda i: (0, 0)),
      out_shape=jax.ShapeDtypeStruct((ngraphs, dout), jnp.float32),
      scratch_shapes=[
          pltpu.VMEM((ngraphs, dout), jnp.float32),
          pltpu.VMEM((ngraphs,), jnp.float32),
      ],
  )(a2a, a2b, d0, d1, b2, batchp)


def kernel(x, edge_index, batch, W1, b1, W2, b2):
  n = x.shape[0]
  e = edge_index.shape[1]
  ngraphs = 64

  npad = ((n + 64 + 255) // 256) * 256
  ecell = NC * NS * CK * WIN
  epad = ((e + ecell - 1) // ecell) * ecell
  spread = 64
  pad_idx = n + (jnp.arange(epad - e, dtype=jnp.int32) % spread)
  src2d = jnp.concatenate([edge_index[0], pad_idx]).reshape(epad // CK, CK)
  dst2d = jnp.concatenate([edge_index[1], pad_idx]).reshape(epad // CK, CK)
  xp = jnp.pad(x, ((0, npad - n), (0, 0)))
  batchp = jnp.pad(batch, (0, npad - n), constant_values=ngraphs)

  degp = _deg_call(dst2d, npad)
  d0, d1 = degp[0], degp[1]
  h1p = _tc1_call(xp, W1, d0, d1)
  a1 = _agg1_call(h1p, src2d, dst2d, npad)
  h2a, h2b = _tc2_call(a1[0], a1[1], h1p, d0, d1, b1, W2)
  a2a, a2b = _agg2_call(h2a, h2b, src2d, dst2d, npad)
  return _tc3_call(a2a, a2b, d0, d1, b2, batchp, ngraphs)


# speedup vs baseline: 1.0156x; 1.0043x over previous
import functools

import jax
import jax.numpy as jnp
from jax import lax
from jax.experimental import pallas as pl
from jax.experimental.pallas import tpu as pltpu
from jax.experimental.pallas import tpu_sc as plsc

NC = 2
NS = 16
LN = 16
CK = 64
BN = 256


def _mesh():
  return plsc.VectorSubcoreMesh(
      core_axis_name="c", subcore_axis_name="s", num_cores=NC,
      num_subcores=NS)


def _deg_call(dst2d, npad):
  er = dst2d.shape[0]
  rpt = er // (NC * NS)
  npt = npad // NS

  @functools.partial(
      pl.kernel,
      out_type=jax.ShapeDtypeStruct((NC, npad), jnp.float32),
      mesh=_mesh(),
      scratch_types=[
          pltpu.VMEM((rpt, CK), jnp.int32),
          pltpu.VMEM((CK,), jnp.float32),
          pltpu.VMEM((npt,), jnp.float32),
          pltpu.VMEM_SHARED((npad,), jnp.float32),
          pltpu.SemaphoreType.DMA,
      ])
  def k(dst_hbm, out_hbm, idx_v, ones_v, zero_v, acc, sem):
    c = lax.axis_index("c")
    s = lax.axis_index("s")
    tid = c * NS + s

    def fill_zero(r, carry):
      zero_v[pl.ds(r * LN, LN)] = jnp.zeros((LN,), jnp.float32)
      return carry

    lax.fori_loop(0, npt // LN, fill_zero, 0)
    for j in range(CK // LN):
      ones_v[pl.ds(j * LN, LN)] = jnp.ones((LN,), jnp.float32)

    pltpu.sync_copy(zero_v, acc.at[pl.ds(s * npt, npt)])
    pltpu.sync_copy(dst_hbm.at[pl.ds(tid * rpt, rpt)], idx_v)
    plsc.subcore_barrier()

    def deg_body(j, carry):
      pltpu.async_copy(ones_v, acc.at[idx_v.at[j]], sem, add=True)
      return carry

    lax.fori_loop(0, rpt, deg_body, 0)

    def deg_drain(j, carry):
      pltpu.make_async_copy(ones_v, acc.at[idx_v.at[j]], sem).wait()
      return carry

    lax.fori_loop(0, rpt, deg_drain, 0)
    plsc.subcore_barrier()
    pltpu.sync_copy(acc.at[pl.ds(s * npt, npt)],
                    out_hbm.at[c, pl.ds(s * npt, npt)])

  return k(dst2d)


WIN = 40


NBUF = 4


def _edge_pipe(h_hbm, src_hbm, dst_hbm, base, nwin,
               src_v, dst_v, rows, gsems, ssems, acc):
  def win_body(w, carry):
    pltpu.sync_copy(src_hbm.at[pl.ds(base + w * WIN, WIN)], src_v)
    pltpu.sync_copy(dst_hbm.at[pl.ds(base + w * WIN, WIN)], dst_v)
    pltpu.async_copy(h_hbm.at[src_v.at[0]], rows[0], gsems[0])
    pltpu.async_copy(h_hbm.at[src_v.at[1]], rows[1], gsems[1])

    def body(j4, c4):
      j = j4 * NBUF
      for b in range(NBUF):
        jj = j + b
        bn = (b + 2) % NBUF
        pltpu.make_async_copy(h_hbm.at[src_v.at[jj]], rows[b],
                              gsems[b]).wait()

        @pl.when(jj >= 2)
        def _():
          pltpu.make_async_copy(rows[bn], acc.at[dst_v.at[jj]],
                                ssems[bn]).wait()

        @pl.when(jj + 2 < WIN)
        def _():
          pltpu.async_copy(h_hbm.at[src_v.at[jj + 2]], rows[bn], gsems[bn])

        pltpu.async_copy(rows[b], acc.at[dst_v.at[jj]], ssems[b], add=True)
      return c4

    lax.fori_loop(0, WIN // NBUF, body, 0)
    for jj in (WIN - 2, WIN - 1):
      b = jj % NBUF
      pltpu.make_async_copy(rows[b], acc.at[dst_v.at[jj]], ssems[b]).wait()
    return carry

  lax.fori_loop(0, nwin, win_body, 0)


def _agg1_call(h1p, src2d, dst2d, npad):
  er = src2d.shape[0]
  rpt = er // (NC * NS)
  npt = npad // NS
  h = h1p.shape[1]

  @functools.partial(
      pl.kernel,
      out_type=jax.ShapeDtypeStruct((NC, npad, h), jnp.float32),
      mesh=_mesh(),
      scratch_types=[
          pltpu.VMEM((WIN, CK), jnp.int32),
          pltpu.VMEM((WIN, CK), jnp.int32),
      ] + [pltpu.VMEM((CK, h), jnp.float32)] * NBUF
        + [pltpu.VMEM_SHARED((npad, h), jnp.float32)]
        + [pltpu.SemaphoreType.DMA] * (2 * NBUF))
  def k(h_hbm, src_hbm, dst_hbm, out_hbm, src_v, dst_v, r0, r1, r2, r3,
        acc, g0, g1, g2, g3, s0, s1, s2, s3):
    rows = (r0, r1, r2, r3)
    gsems = (g0, g1, g2, g3)
    ssems = (s0, s1, s2, s3)
    c = lax.axis_index("c")
    s = lax.axis_index("s")
    tid = c * NS + s
    pltpu.sync_copy(h_hbm.at[pl.ds(s * npt, npt)],
                    acc.at[pl.ds(s * npt, npt)])
    plsc.subcore_barrier()
    _edge_pipe(h_hbm, src_hbm, dst_hbm, tid * rpt, rpt // WIN,
               src_v, dst_v, rows, gsems, ssems, acc)
    plsc.subcore_barrier()
    pltpu.sync_copy(acc.at[pl.ds(s * npt, npt)],
                    out_hbm.at[c, pl.ds(s * npt, npt)])

  return k(h1p, src2d, dst2d)


def _agg2_call(h2a, h2b, src2d, dst2d, npad):
  er = src2d.shape[0]
  rpt = er // NS
  npt = npad // NS
  h = h2a.shape[1]

  @functools.partial(
      pl.kernel,
      out_type=(jax.ShapeDtypeStruct((npad, h), jnp.float32),
                jax.ShapeDtypeStruct((npad, h), jnp.float32)),
      mesh=_mesh(),
      scratch_types=[
          pltpu.VMEM((WIN, CK), jnp.int32),
          pltpu.VMEM((WIN, CK), jnp.int32),
      ] + [pltpu.VMEM((CK, h), jnp.float32)] * NBUF
        + [pltpu.VMEM_SHARED((npad, h), jnp.float32)]
        + [pltpu.SemaphoreType.DMA] * (2 * NBUF))
  def k(ha_hbm, hb_hbm, src_hbm, dst_hbm, outa_hbm, outb_hbm,
        src_v, dst_v, r0, r1, r2, r3,
        acc, g0, g1, g2, g3, s0, s1, s2, s3):
    rows = (r0, r1, r2, r3)
    gsems = (g0, g1, g2, g3)
    ssems = (s0, s1, s2, s3)
    c = lax.axis_index("c")
    s = lax.axis_index("s")
    for ci, h_hbm, out_hbm in ((0, ha_hbm, outa_hbm), (1, hb_hbm, outb_hbm)):

      @pl.when(c == ci)
      def _():
        pltpu.sync_copy(h_hbm.at[pl.ds(s * npt, npt)],
                        acc.at[pl.ds(s * npt, npt)])
        plsc.subcore_barrier()
        _edge_pipe(h_hbm, src_hbm, dst_hbm, s * rpt, rpt // WIN,
                   src_v, dst_v, rows, gsems, ssems, acc)
        plsc.subcore_barrier()
        pltpu.sync_copy(acc.at[pl.ds(s * npt, npt)],
                        out_hbm.at[pl.ds(s * npt, npt)])

  return k(h2a, h2b, src2d, dst2d)


def _tc1_call(xp, w1, d0, d1):
  npad, din = xp.shape
  dh = w1.shape[1]
  grid = (npad // BN,)

  def body(x_ref, w_ref, d0_ref, d1_ref, o_ref):
    dinv = lax.rsqrt(1.0 + d0_ref[...] + d1_ref[...])
    o_ref[...] = jnp.dot(x_ref[...], w_ref[...],
                         preferred_element_type=jnp.float32) * dinv[:, None]

  return pl.pallas_call(
      body,
      grid=grid,
      in_specs=[
          pl.BlockSpec((BN, din), lambda i: (i, 0)),
          pl.BlockSpec((din, dh), lambda i: (0, 0)),
          pl.BlockSpec((BN,), lambda i: (i,)),
          pl.BlockSpec((BN,), lambda i: (i,)),
      ],
      out_specs=pl.BlockSpec((BN, dh), lambda i: (i, 0)),
      out_shape=jax.ShapeDtypeStruct((npad, dh), jnp.float32),
  )(xp, w1, d0, d1)


def _tc2_call(a1a, a1b, h1p, d0, d1, b1, w2):
  npad, dh = h1p.shape
  dout = w2.shape[1]
  hh = dout // 2
  grid = (npad // BN,)

  def body(aa_ref, ab_ref, hp_ref, d0_ref, d1_ref, b1_ref, w_ref,
           oa_ref, ob_ref):
    dinv = lax.rsqrt(1.0 + d0_ref[...] + d1_ref[...])
    agg = aa_ref[...] + ab_ref[...] - hp_ref[...]
    t = jnp.maximum(agg * dinv[:, None] + b1_ref[...][None, :], 0.0)
    h2p = jnp.dot(t, w_ref[...],
                  preferred_element_type=jnp.float32) * dinv[:, None]
    oa_ref[...] = h2p[:, :hh]
    ob_ref[...] = h2p[:, hh:]

  return pl.pallas_call(
      body,
      grid=grid,
      in_specs=[
          pl.BlockSpec((BN, dh), lambda i: (i, 0)),
          pl.BlockSpec((BN, dh), lambda i: (i, 0)),
          pl.BlockSpec((BN, dh), lambda i: (i, 0)),
          pl.BlockSpec((BN,), lambda i: (i,)),
          pl.BlockSpec((BN,), lambda i: (i,)),
          pl.BlockSpec((dh,), lambda i: (0,)),
          pl.BlockSpec((dh, dout), lambda i: (0, 0)),
      ],
      out_specs=[
          pl.BlockSpec((BN, hh), lambda i: (i, 0)),
          pl.BlockSpec((BN, hh), lambda i: (i, 0)),
      ],
      out_shape=(jax.ShapeDtypeStruct((npad, hh), jnp.float32),
                 jax.ShapeDtypeStruct((npad, hh), jnp.float32)),
  )(a1a, a1b, h1p, d0, d1, b1, w2)


def _tc3_call(a2a, a2b, d0, d1, b2, batchp, ngraphs):
  npad, hh = a2a.shape
  dout = 2 * hh
  grid = (npad // BN,)
  nsteps = npad // BN

  def body(aa_ref, ab_ref, d0_ref, d1_ref, b2_ref, bt_ref, o_ref,
           accp, cnt):
    i = pl.program_id(0)

    @pl.when(i == 0)
    def _():
      accp[...] = jnp.zeros_like(accp)
      cnt[...] = jnp.zeros_like(cnt)

    dinv = lax.rsqrt(1.0 + d0_ref[...] + d1_ref[...])
    h2 = (jnp.concatenate([aa_ref[...], ab_ref[...]], axis=1)
          * dinv[:, None] + b2_ref[...][None, :])
    gids = lax.broadcasted_iota(jnp.int32, (1, ngraphs), 1)
    oh = (bt_ref[...][:, None] == gids).astype(jnp.float32)
    accp[...] += lax.dot_general(oh, h2, (((0,), (0,)), ((), ())),
                                 preferred_element_type=jnp.float32)
    cnt[...] += jnp.sum(oh, axis=0)

    @pl.when(i == nsteps - 1)
    def _():
      pooled = accp[...] / jnp.maximum(cnt[...], 1.0)[:, None]
      m = jnp.max(pooled, axis=1, keepdims=True)
      lse = jnp.log(jnp.sum(jnp.exp(pooled - m), axis=1, keepdims=True))
      o_ref[...] = pooled - m - lse

  return pl.pallas_call(
      body,
      grid=grid,
      in_specs=[
          pl.BlockSpec((BN, hh), lambda i: (i, 0)),
          pl.BlockSpec((BN, hh), lambda i: (i, 0)),
          pl.BlockSpec((BN,), lambda i: (i,)),
          pl.BlockSpec((BN,), lambda i: (i,)),
          pl.BlockSpec((dout,), lambda i: (0,)),
          pl.BlockSpec((BN,), lambda i: (i,)),
      ],
      out_specs=pl.BlockSpec((ngraphs, dout), lambda i: (0, 0)),
      out_shape=jax.ShapeDtypeStruct((ngraphs, dout), jnp.float32),
      scratch_shapes=[
          pltpu.VMEM((ngraphs, dout), jnp.float32),
          pltpu.VMEM((ngraphs,), jnp.float32),
      ],
  )(a2a, a2b, d0, d1, b2, batchp)


def kernel(x, edge_index, batch, W1, b1, W2, b2):
  n = x.shape[0]
  e = edge_index.shape[1]
  ngraphs = 64

  npad = ((n + 64 + 255) // 256) * 256
  ecell = NC * NS * CK * WIN
  epad = ((e + ecell - 1) // ecell) * ecell
  spread = 64
  pad_idx = n + (jnp.arange(epad - e, dtype=jnp.int32) % spread)
  src2d = jnp.concatenate([edge_index[0], pad_idx]).reshape(epad // CK, CK)
  dst2d = jnp.concatenate([edge_index[1], pad_idx]).reshape(epad // CK, CK)
  xp = jnp.pad(x, ((0, npad - n), (0, 0)))
  batchp = jnp.pad(batch, (0, npad - n), constant_values=ngraphs)

  degp = _deg_call(dst2d, npad)
  d0, d1 = degp[0], degp[1]
  h1p = _tc1_call(xp, W1, d0, d1)
  a1 = _agg1_call(h1p, src2d, dst2d, npad)
  h2a, h2b = _tc2_call(a1[0], a1[1], h1p, d0, d1, b1, W2)
  a2a, a2b = _agg2_call(h2a, h2b, src2d, dst2d, npad)
  return _tc3_call(a2a, a2b, d0, d1, b2, batchp, ngraphs)
